# Initial kernel scaffold; baseline (speedup 1.0000x reference)
import jax
import jax.numpy as jnp
from jax.experimental import pallas as pl


def kernel(mel, mel_lengths, decoder_inputs, start_embedding, l1f_Wih, l1f_Whh, l1f_b, l1b_Wih, l1b_Whh, l1b_b, l2f_Wih, l2f_Whh, l2f_b, l2b_Wih, l2b_Whh, l2b_b, dec_Wih, dec_Whh, dec_b, q_W, m_W, loc_conv_W, loc_W, v_W, ph_W, ph_b, ps_W, ps_b):
    raise NotImplementedError("write your pallas kernel here")


# speedup vs baseline: 4.2889x; 4.2889x over previous
import functools

import jax
import jax.numpy as jnp
from jax.experimental import pallas as pl
from jax.experimental.pallas import tpu as pltpu


def _lstm_kernel(len_ref, x_ref, wih_ref, whh_ref, b_ref, out_ref, h_ref, c_ref):
    d = pl.program_id(0)
    t = pl.program_id(1)
    T = pl.num_programs(1)

    @pl.when(t == 0)
    def _():
        h_ref[...] = jnp.zeros_like(h_ref)
        c_ref[...] = jnp.zeros_like(c_ref)

    t_actual = jnp.where(d == 0, t, T - 1 - t)
    hd = h_ref.shape[1]
    x = x_ref[0]
    g = (jnp.dot(x, wih_ref[0], preferred_element_type=jnp.float32)
         + jnp.dot(h_ref[...], whh_ref[0], preferred_element_type=jnp.float32)
         + b_ref[0])
    i = jax.nn.sigmoid(g[:, 0 * hd:1 * hd])
    f = jax.nn.sigmoid(g[:, 1 * hd:2 * hd])
    gg = jnp.tanh(g[:, 2 * hd:3 * hd])
    o = jax.nn.sigmoid(g[:, 3 * hd:4 * hd])
    c_new = f * c_ref[...] + i * gg
    h_new = o * jnp.tanh(c_new)
    m = len_ref[...] > t_actual
    h_ref[...] = jnp.where(m, h_new, h_ref[...])
    c_ref[...] = jnp.where(m, c_new, c_ref[...])
    out_ref[0, 0] = jnp.where(m, h_new, 0.0)


def _bilstm_layer(x_tm, lengths, wih_f, whh_f, b_f, wih_b, whh_b, b_b):
    T, B, D = x_tm.shape
    HD4 = wih_f.shape[0]
    HD = HD4 // 4
    wih = jnp.stack([wih_f.T, wih_b.T])
    whh = jnp.stack([whh_f.T, whh_b.T])
    b = jnp.stack([b_f[None, :], b_b[None, :]])
    len2 = lengths.astype(jnp.int32)[:, None]

    def t_idx(d, t):
        return jnp.where(d == 0, t, T - 1 - t)

    return pl.pallas_call(
        _lstm_kernel,
        out_shape=jax.ShapeDtypeStruct((2, T, B, HD), jnp.float32),
        grid=(2, T),
        in_specs=[
            pl.BlockSpec((B, 1), lambda d, t: (0, 0)),
            pl.BlockSpec((1, B, D), lambda d, t: (t_idx(d, t), 0, 0)),
            pl.BlockSpec((1, D, HD4), lambda d, t: (d, 0, 0)),
            pl.BlockSpec((1, HD, HD4), lambda d, t: (d, 0, 0)),
            pl.BlockSpec((1, 1, HD4), lambda d, t: (d, 0, 0)),
        ],
        out_specs=pl.BlockSpec((1, 1, B, HD), lambda d, t: (d, t_idx(d, t), 0, 0)),
        scratch_shapes=[
            pltpu.VMEM((B, HD), jnp.float32),
            pltpu.VMEM((B, HD), jnp.float32),
        ],
        compiler_params=pltpu.CompilerParams(
            dimension_semantics=("arbitrary", "arbitrary"),
        ),
        name="bilstm_layer",
    )(len2, x_tm, wih, whh, b)


def _decoder_kernel(K, A, n_chunk,
                    len_ref, dec_ref, mem_ref, mW_ref, wih_ref, whh_ref, b_ref,
                    qW_ref, M2_ref, v_ref, phW_ref, phb_ref, psW_ref, psb_ref,
                    hid_out, logit_out, align_out,
                    h_ref, c_ref, ctx_ref, awp_ref, awcp_ref, U_ref, pm_ref):
    t = pl.program_id(0)
    B, Tm, H = mem_ref.shape
    HD = h_ref.shape[1]
    pad = (K - 1) // 2

    @pl.when(t == 0)
    def _():
        h_ref[...] = jnp.zeros_like(h_ref)
        c_ref[...] = jnp.zeros_like(c_ref)
        ctx_ref[...] = jnp.zeros_like(ctx_ref)
        awp_ref[...] = jnp.zeros_like(awp_ref)
        awcp_ref[...] = jnp.zeros_like(awcp_ref)
        U_ref[...] = jnp.zeros_like(U_ref)
        pm_ref[...] = jax.lax.dot_general(
            mW_ref[...].astype(jnp.bfloat16), mem_ref[...],
            (((1,), (2,)), ((), ())),
            preferred_element_type=jnp.float32).astype(jnp.bfloat16)

    cell_in = jnp.concatenate([dec_ref[0], ctx_ref[...]],
                              axis=1).astype(jnp.bfloat16)
    g = (jnp.dot(cell_in, wih_ref[...], preferred_element_type=jnp.float32)
         + jnp.dot(h_ref[...].astype(jnp.bfloat16), whh_ref[...],
                   preferred_element_type=jnp.float32)
         + b_ref[...])
    ig = jax.nn.sigmoid(g[:, 0 * HD:1 * HD])
    fg = jax.nn.sigmoid(g[:, 1 * HD:2 * HD])
    gg = jnp.tanh(g[:, 2 * HD:3 * HD])
    og = jax.nn.sigmoid(g[:, 3 * HD:4 * HD])
    c_new = fg * c_ref[...] + ig * gg
    h_new = og * jnp.tanh(c_new)
    h_ref[...] = h_new
    c_ref[...] = c_new

    for k in range(K):
        U_ref[k] = awp_ref[:, k:k + Tm].astype(jnp.bfloat16)
        U_ref[K + 1 + k] = awcp_ref[:, k:k + Tm].astype(jnp.bfloat16)

    q = jnp.dot(h_new, qW_ref[...], preferred_element_type=jnp.float32)
    q_t = q.T[:, :, None]
    v_col = v_ref[...].reshape(A)[:, None, None]

    e_parts = []
    cs = B // n_chunk
    for ci in range(n_chunk):
        sl = slice(ci * cs, (ci + 1) * cs)
        u_c = U_ref[:, sl, :]
        loc_c = jax.lax.dot_general(
            M2_ref[...], u_c, (((1,), (0,)), ((), ())),
            preferred_element_type=jnp.float32)
        x_c = loc_c + pm_ref[:, sl, :].astype(jnp.float32) + q_t[:, sl]
        e_parts.append(jnp.sum(v_col * jnp.tanh(x_c), axis=0))
    e = jnp.concatenate(e_parts, axis=0)

    tpos = jax.lax.broadcasted_iota(jnp.int32, (1, Tm), 1)
    e = jnp.where(tpos >= len_ref[...], -1e9, e)
    e = e - jnp.max(e, axis=1, keepdims=True)
    ex = jnp.exp(e)
    aw_new = ex / jnp.sum(ex, axis=1, keepdims=True)

    ctx_new = jnp.sum(aw_new[:, :, None] * mem_ref[...].astype(jnp.float32),
                      axis=1)
    ctx_ref[...] = ctx_new
    awp_ref[:, pad:pad + Tm] = aw_new
    awcp_ref[:, pad:pad + Tm] = awcp_ref[:, pad:pad + Tm] + aw_new

    hid = jnp.maximum(
        jnp.dot(jnp.concatenate([h_new, ctx_new], axis=1).astype(jnp.bfloat16),
                phW_ref[...],
                preferred_element_type=jnp.float32) + phb_ref[...], 0.0)
    logit = jnp.dot(hid, psW_ref[...], preferred_element_type=jnp.float32) + psb_ref[...]

    hid_out[0] = hid
    logit_out[0] = logit
    align_out[0] = aw_new


def _decoder(dec_seq, mem_len, memory, dec_Wih, dec_Whh, dec_b,
             q_W, m_W, loc_conv_W, loc_W, v_W, ph_W, ph_b, ps_W, ps_b):
    TD, B, EMB = dec_seq.shape
    Tm = memory.shape[1]
    H = memory.shape[2]
    HD = dec_Whh.shape[1]
    A = q_W.shape[0]
    NF, _, K = loc_conv_W.shape
    NSYM1 = ps_W.shape[0]
    pad = (K - 1) // 2
    TP = Tm + 2 * pad + (-(Tm + 2 * pad)) % 16

    M = jnp.einsum('af,fck->ack', loc_W, loc_conv_W)
    M2 = jnp.zeros((A, 2 * K + 2), jnp.float32)
    M2 = M2.at[:, 0:K].set(M[:, 0]).at[:, K + 1:2 * K + 1].set(M[:, 1])

    mem_bf16 = memory.astype(jnp.bfloat16)
    n_chunk = min(8, B)

    kern = functools.partial(_decoder_kernel, K, A, n_chunk)
    hid, logit, align = pl.pallas_call(
        kern,
        out_shape=(
            jax.ShapeDtypeStruct((TD, B, H), jnp.float32),
            jax.ShapeDtypeStruct((TD, B, NSYM1), jnp.float32),
            jax.ShapeDtypeStruct((TD, B, Tm), jnp.float32),
        ),
        grid=(TD,),
        in_specs=[
            pl.BlockSpec((B, 1), lambda t: (0, 0)),
            pl.BlockSpec((1, B, EMB), lambda t: (t, 0, 0)),
            pl.BlockSpec((B, Tm, H), lambda t: (0, 0, 0)),
            pl.BlockSpec((A, H), lambda t: (0, 0)),
            pl.BlockSpec((EMB + H, 4 * HD), lambda t: (0, 0)),
            pl.BlockSpec((HD, 4 * HD), lambda t: (0, 0)),
            pl.BlockSpec((1, 4 * HD), lambda t: (0, 0)),
            pl.BlockSpec((HD, A), lambda t: (0, 0)),
            pl.BlockSpec((A, 2 * K + 2), lambda t: (0, 0)),
            pl.BlockSpec((1, A), lambda t: (0, 0)),
            pl.BlockSpec((H + H, H), lambda t: (0, 0)),
            pl.BlockSpec((1, H), lambda t: (0, 0)),
            pl.BlockSpec((H, NSYM1), lambda t: (0, 0)),
            pl.BlockSpec((1, NSYM1), lambda t: (0, 0)),
        ],
        out_specs=(
            pl.BlockSpec((1, B, H), lambda t: (t, 0, 0)),
            pl.BlockSpec((1, B, NSYM1), lambda t: (t, 0, 0)),
            pl.BlockSpec((1, B, Tm), lambda t: (t, 0, 0)),
        ),
        scratch_shapes=[
            pltpu.VMEM((B, HD), jnp.float32),
            pltpu.VMEM((B, HD), jnp.float32),
            pltpu.VMEM((B, H), jnp.float32),
            pltpu.VMEM((B, TP), jnp.float32),
            pltpu.VMEM((B, TP), jnp.float32),
            pltpu.VMEM((2 * K + 2, B, Tm), jnp.bfloat16),
            pltpu.VMEM((A, B, Tm), jnp.bfloat16),
        ],
        compiler_params=pltpu.CompilerParams(
            dimension_semantics=("arbitrary",),
            vmem_limit_bytes=61440000,
        ),
        name="decoder",
    )(mem_len.astype(jnp.int32)[:, None], dec_seq, mem_bf16, m_W,
      dec_Wih.T.astype(jnp.bfloat16), dec_Whh.T.astype(jnp.bfloat16),
      dec_b[None, :], q_W.T, M2.astype(jnp.bfloat16), v_W,
      ph_W.T.astype(jnp.bfloat16), ph_b[None, :], ps_W.T, ps_b[None, :])
    return hid, logit, align


def kernel(mel, mel_lengths, decoder_inputs, start_embedding, l1f_Wih, l1f_Whh,
           l1f_b, l1b_Wih, l1b_Whh, l1b_b, l2f_Wih, l2f_Whh, l2f_b, l2b_Wih,
           l2b_Whh, l2b_b, dec_Wih, dec_Whh, dec_b, q_W, m_W, loc_conv_W,
           loc_W, v_W, ph_W, ph_b, ps_W, ps_b):
    B, NMEL, T = mel.shape
    lengths = mel_lengths.astype(jnp.int32)

    x_tm = mel.transpose(2, 0, 1)
    out1 = _bilstm_layer(x_tm, lengths, l1f_Wih, l1f_Whh, l1f_b,
                         l1b_Wih, l1b_Whh, l1b_b)
    HD = out1.shape[3]
    cat1 = jnp.concatenate([out1[0], out1[1]], axis=2)
    x2 = cat1.transpose(1, 0, 2).reshape(B, T // 2, 4 * HD).transpose(1, 0, 2)

    mem_len = jnp.ceil(lengths.astype(jnp.float32) / 2).astype(jnp.int32)
    out2 = _bilstm_layer(x2, mem_len, l2f_Wih, l2f_Whh, l2f_b,
                         l2b_Wih, l2b_Whh, l2b_b)
    memory = jnp.concatenate([out2[0], out2[1]], axis=2).transpose(1, 0, 2)

    dec_seq = jnp.concatenate(
        [start_embedding[None], decoder_inputs.transpose(2, 0, 1)], axis=0)

    hid, logit, align = _decoder(dec_seq, mem_len, memory, dec_Wih, dec_Whh,
                                 dec_b, q_W, m_W, loc_conv_W, loc_W, v_W,
                                 ph_W, ph_b, ps_W, ps_b)
    return (hid.transpose(1, 0, 2), logit.transpose(1, 0, 2),
            align.transpose(1, 0, 2))


# speedup vs baseline: 4.3000x; 1.0026x over previous
import functools

import jax
import jax.numpy as jnp
from jax.experimental import pallas as pl
from jax.experimental.pallas import tpu as pltpu


def _inproj_kernel(x_ref, w_ref, out_ref):
    out_ref[...] = jnp.dot(
        x_ref[...].astype(jnp.bfloat16), w_ref[...],
        preferred_element_type=jnp.float32).astype(jnp.bfloat16)


def _inproj(x_flat, w2):
    M, D = x_flat.shape
    N = w2.shape[1]
    BM = 1024 if M % 1024 == 0 else M
    return pl.pallas_call(
        _inproj_kernel,
        out_shape=jax.ShapeDtypeStruct((M, N), jnp.bfloat16),
        grid=(M // BM,),
        in_specs=[
            pl.BlockSpec((BM, D), lambda i: (i, 0)),
            pl.BlockSpec((D, N), lambda i: (0, 0)),
        ],
        out_specs=pl.BlockSpec((BM, N), lambda i: (i, 0)),
        compiler_params=pltpu.CompilerParams(
            dimension_semantics=("arbitrary",),
        ),
        name="lstm_inproj",
    )(x_flat, w2)


def _lstm_kernel(xproj, len_ref, x_ref, wih_ref, whh_ref, b_ref, out_ref,
                 h_ref, c_ref):
    d = pl.program_id(0)
    t = pl.program_id(1)
    T = pl.num_programs(1)

    @pl.when(t == 0)
    def _():
        h_ref[...] = jnp.zeros_like(h_ref)
        c_ref[...] = jnp.zeros_like(c_ref)

    t_actual = jnp.where(d == 0, t, T - 1 - t)
    hd = h_ref.shape[1]
    if xproj:
        gx = jnp.dot(x_ref[0, 0], wih_ref[0],
                     preferred_element_type=jnp.float32)
    else:
        gx = x_ref[0, 0].astype(jnp.float32)
    g = (gx
         + jnp.dot(h_ref[...], whh_ref[0], preferred_element_type=jnp.float32)
         + b_ref[0])
    i = jax.nn.sigmoid(g[:, 0 * hd:1 * hd])
    f = jax.nn.sigmoid(g[:, 1 * hd:2 * hd])
    gg = jnp.tanh(g[:, 2 * hd:3 * hd])
    o = jax.nn.sigmoid(g[:, 3 * hd:4 * hd])
    c_new = f * c_ref[...] + i * gg
    h_new = o * jnp.tanh(c_new)
    m = len_ref[...] > t_actual
    h_ref[...] = jnp.where(m, h_new, h_ref[...])
    c_ref[...] = jnp.where(m, c_new, c_ref[...])
    out_ref[0] = jnp.where(m, h_new, 0.0)


def _bilstm_layer(x_tm, lengths, wih_f, whh_f, b_f, wih_b, whh_b, b_b,
                  hoist_inproj):
    T, B, D = x_tm.shape
    HD4 = wih_f.shape[0]
    HD = HD4 // 4

    if hoist_inproj:
        wih2 = jnp.concatenate([wih_f.T, wih_b.T], axis=1).astype(jnp.bfloat16)
        gflat = _inproj(x_tm.reshape(T * B, D), wih2)
        xin = gflat.reshape(T, B, 2, HD4).transpose(2, 0, 1, 3)
        wih = jnp.zeros((2, 1, HD4), jnp.float32)
        DI = HD4
    else:
        xin = jnp.broadcast_to(x_tm[None], (2, T, B, D))
        wih = jnp.stack([wih_f.T, wih_b.T])
        DI = D

    whh = jnp.stack([whh_f.T, whh_b.T])
    b = jnp.stack([b_f[None, :], b_b[None, :]])
    len2 = lengths.astype(jnp.int32)[:, None]

    def t_idx(d, t):
        return jnp.where(d == 0, t, T - 1 - t)

    kern = functools.partial(_lstm_kernel, not hoist_inproj)
    return pl.pallas_call(
        kern,
        out_shape=jax.ShapeDtypeStruct((T, B, 2 * HD), jnp.float32),
        grid=(2, T),
        in_specs=[
            pl.BlockSpec((B, 1), lambda d, t: (0, 0)),
            pl.BlockSpec((1, 1, B, DI), lambda d, t: (d, t_idx(d, t), 0, 0)),
            pl.BlockSpec((1, wih.shape[1], HD4), lambda d, t: (d, 0, 0)),
            pl.BlockSpec((1, HD, HD4), lambda d, t: (d, 0, 0)),
            pl.BlockSpec((1, 1, HD4), lambda d, t: (d, 0, 0)),
        ],
        out_specs=pl.BlockSpec((1, B, HD), lambda d, t: (t_idx(d, t), 0, d)),
        scratch_shapes=[
            pltpu.VMEM((B, HD), jnp.float32),
            pltpu.VMEM((B, HD), jnp.float32),
        ],
        compiler_params=pltpu.CompilerParams(
            dimension_semantics=("arbitrary", "arbitrary"),
        ),
        name="bilstm_layer",
    )(len2, xin, wih, whh, b)


def _decoder_kernel(K, A, n_chunk,
                    len_ref, dec_ref, mem_ref, mW_ref, wih_ref, whh_ref, b_ref,
                    qW_ref, M2_ref, v_ref, phW_ref, phb_ref, psW_ref, psb_ref,
                    hid_out, logit_out, align_out,
                    h_ref, c_ref, ctx_ref, awp_ref, awcp_ref, U_ref, pm_ref):
    t = pl.program_id(1)
    B, Tm, H = mem_ref.shape
    HD = h_ref.shape[1]
    pad = (K - 1) // 2

    @pl.when(t == 0)
    def _():
        h_ref[...] = jnp.zeros_like(h_ref)
        c_ref[...] = jnp.zeros_like(c_ref)
        ctx_ref[...] = jnp.zeros_like(ctx_ref)
        awp_ref[...] = jnp.zeros_like(awp_ref)
        awcp_ref[...] = jnp.zeros_like(awcp_ref)
        U_ref[...] = jnp.zeros_like(U_ref)
        for cj in range(n_chunk):
            sl0 = slice(cj * (B // n_chunk), (cj + 1) * (B // n_chunk))
            pm_ref[:, sl0, :] = jax.lax.dot_general(
                mW_ref[...].astype(jnp.bfloat16), mem_ref[sl0, :, :],
                (((1,), (2,)), ((), ())),
                preferred_element_type=jnp.float32).astype(jnp.bfloat16)

    cell_in = jnp.concatenate([dec_ref[0], ctx_ref[...]],
                              axis=1).astype(jnp.bfloat16)
    g = (jnp.dot(cell_in, wih_ref[...], preferred_element_type=jnp.float32)
         + jnp.dot(h_ref[...].astype(jnp.bfloat16), whh_ref[...],
                   preferred_element_type=jnp.float32)
         + b_ref[...])
    ig = jax.nn.sigmoid(g[:, 0 * HD:1 * HD])
    fg = jax.nn.sigmoid(g[:, 1 * HD:2 * HD])
    gg = jnp.tanh(g[:, 2 * HD:3 * HD])
    og = jax.nn.sigmoid(g[:, 3 * HD:4 * HD])
    c_new = fg * c_ref[...] + ig * gg
    h_new = og * jnp.tanh(c_new)
    h_ref[...] = h_new
    c_ref[...] = c_new

    for k in range(K):
        U_ref[k] = awp_ref[:, k:k + Tm].astype(jnp.bfloat16)
        U_ref[K + 1 + k] = awcp_ref[:, k:k + Tm].astype(jnp.bfloat16)

    q = jnp.dot(h_new, qW_ref[...], preferred_element_type=jnp.float32)
    q_t = q.T[:, :, None]
    v_col = v_ref[...].reshape(A)[:, None, None]

    e_parts = []
    cs = B // n_chunk
    for ci in range(n_chunk):
        sl = slice(ci * cs, (ci + 1) * cs)
        u_c = U_ref[:, sl, :]
        loc_c = jax.lax.dot_general(
            M2_ref[...], u_c, (((1,), (0,)), ((), ())),
            preferred_element_type=jnp.float32)
        x_c = loc_c + pm_ref[:, sl, :].astype(jnp.float32) + q_t[:, sl]
        e_parts.append(jnp.sum(v_col * jnp.tanh(x_c), axis=0))
    e = jnp.concatenate(e_parts, axis=0)

    tpos = jax.lax.broadcasted_iota(jnp.int32, (1, Tm), 1)
    e = jnp.where(tpos >= len_ref[...], -1e9, e)
    e = e - jnp.max(e, axis=1, keepdims=True)
    ex = jnp.exp(e)
    aw_new = ex / jnp.sum(ex, axis=1, keepdims=True)

    ctx_new = jnp.sum(aw_new[:, :, None] * mem_ref[...].astype(jnp.float32),
                      axis=1)
    ctx_ref[...] = ctx_new
    awp_ref[:, pad:pad + Tm] = aw_new
    awcp_ref[:, pad:pad + Tm] = awcp_ref[:, pad:pad + Tm] + aw_new

    hid = jnp.maximum(
        jnp.dot(jnp.concatenate([h_new, ctx_new], axis=1).astype(jnp.bfloat16),
                phW_ref[...],
                preferred_element_type=jnp.float32) + phb_ref[...], 0.0)
    logit = jnp.dot(hid, psW_ref[...], preferred_element_type=jnp.float32) + psb_ref[...]

    hid_out[0] = hid
    logit_out[0] = logit
    align_out[0] = aw_new


def _decoder(dec_seq, mem_len, memory, dec_Wih, dec_Whh, dec_b,
             q_W, m_W, loc_conv_W, loc_W, v_W, ph_W, ph_b, ps_W, ps_b):
    TD, B, EMB = dec_seq.shape
    Tm = memory.shape[1]
    H = memory.shape[2]
    HD = dec_Whh.shape[1]
    A = q_W.shape[0]
    NF, _, K = loc_conv_W.shape
    NSYM1 = ps_W.shape[0]
    pad = (K - 1) // 2
    TP = Tm + 2 * pad + (-(Tm + 2 * pad)) % 16

    M = jnp.einsum('af,fck->ack', loc_W, loc_conv_W)
    M2 = jnp.zeros((A, 2 * K + 2), jnp.float32)
    M2 = M2.at[:, 0:K].set(M[:, 0]).at[:, K + 1:2 * K + 1].set(M[:, 1])

    mem_bf16 = memory.astype(jnp.bfloat16)
    B2 = B
    NP = 1
    n_chunk = min(8, B2)

    kern = functools.partial(_decoder_kernel, K, A, n_chunk)
    hid, logit, align = pl.pallas_call(
        kern,
        out_shape=(
            jax.ShapeDtypeStruct((TD, B, H), jnp.float32),
            jax.ShapeDtypeStruct((TD, B, NSYM1), jnp.float32),
            jax.ShapeDtypeStruct((TD, B, Tm), jnp.float32),
        ),
        grid=(NP, TD),
        in_specs=[
            pl.BlockSpec((B2, 1), lambda p, t: (p, 0)),
            pl.BlockSpec((1, B2, EMB), lambda p, t: (t, p, 0)),
            pl.BlockSpec((B2, Tm, H), lambda p, t: (p, 0, 0)),
            pl.BlockSpec((A, H), lambda p, t: (0, 0)),
            pl.BlockSpec((EMB + H, 4 * HD), lambda p, t: (0, 0)),
            pl.BlockSpec((HD, 4 * HD), lambda p, t: (0, 0)),
            pl.BlockSpec((1, 4 * HD), lambda p, t: (0, 0)),
            pl.BlockSpec((HD, A), lambda p, t: (0, 0)),
            pl.BlockSpec((A, 2 * K + 2), lambda p, t: (0, 0)),
            pl.BlockSpec((1, A), lambda p, t: (0, 0)),
            pl.BlockSpec((H + H, H), lambda p, t: (0, 0)),
            pl.BlockSpec((1, H), lambda p, t: (0, 0)),
            pl.BlockSpec((H, NSYM1), lambda p, t: (0, 0)),
            pl.BlockSpec((1, NSYM1), lambda p, t: (0, 0)),
        ],
        out_specs=(
            pl.BlockSpec((1, B2, H), lambda p, t: (t, p, 0)),
            pl.BlockSpec((1, B2, NSYM1), lambda p, t: (t, p, 0)),
            pl.BlockSpec((1, B2, Tm), lambda p, t: (t, p, 0)),
        ),
        scratch_shapes=[
            pltpu.VMEM((B2, HD), jnp.float32),
            pltpu.VMEM((B2, HD), jnp.float32),
            pltpu.VMEM((B2, H), jnp.float32),
            pltpu.VMEM((B2, TP), jnp.float32),
            pltpu.VMEM((B2, TP), jnp.float32),
            pltpu.VMEM((2 * K + 2, B2, Tm), jnp.bfloat16),
            pltpu.VMEM((A, B2, Tm), jnp.bfloat16),
        ],
        compiler_params=pltpu.CompilerParams(
            dimension_semantics=("arbitrary", "arbitrary"),
            vmem_limit_bytes=61440000,
        ),
        name="decoder",
    )(mem_len.astype(jnp.int32)[:, None], dec_seq, mem_bf16, m_W,
      dec_Wih.T.astype(jnp.bfloat16), dec_Whh.T.astype(jnp.bfloat16),
      dec_b[None, :], q_W.T, M2.astype(jnp.bfloat16), v_W,
      ph_W.T.astype(jnp.bfloat16), ph_b[None, :], ps_W.T, ps_b[None, :])
    return hid, logit, align


def kernel(mel, mel_lengths, decoder_inputs, start_embedding, l1f_Wih, l1f_Whh,
           l1f_b, l1b_Wih, l1b_Whh, l1b_b, l2f_Wih, l2f_Whh, l2f_b, l2b_Wih,
           l2b_Whh, l2b_b, dec_Wih, dec_Whh, dec_b, q_W, m_W, loc_conv_W,
           loc_W, v_W, ph_W, ph_b, ps_W, ps_b):
    B, NMEL, T = mel.shape
    lengths = mel_lengths.astype(jnp.int32)

    x_tm = mel.transpose(2, 0, 1)
    cat1 = _bilstm_layer(x_tm, lengths, l1f_Wih, l1f_Whh, l1f_b,
                         l1b_Wih, l1b_Whh, l1b_b, False)
    x2 = cat1.transpose(1, 0, 2).reshape(B, T // 2, 2 * cat1.shape[2]) \
             .transpose(1, 0, 2)

    mem_len = jnp.ceil(lengths.astype(jnp.float32) / 2).astype(jnp.int32)
    memory = _bilstm_layer(x2, mem_len, l2f_Wih, l2f_Whh, l2f_b,
                           l2b_Wih, l2b_Whh, l2b_b, True)
    memory = memory.transpose(1, 0, 2)

    dec_seq = jnp.concatenate(
        [start_embedding[None], decoder_inputs.transpose(2, 0, 1)], axis=0)

    hid, logit, align = _decoder(dec_seq, mem_len, memory, dec_Wih, dec_Whh,
                                 dec_b, q_W, m_W, loc_conv_W, loc_W, v_W,
                                 ph_W, ph_b, ps_W, ps_b)
    return (hid.transpose(1, 0, 2), logit.transpose(1, 0, 2),
            align.transpose(1, 0, 2))


# speedup vs baseline: 5.0221x; 1.1679x over previous
import functools

import jax
import jax.numpy as jnp
from jax.experimental import pallas as pl
from jax.experimental.pallas import tpu as pltpu


def _inproj_kernel(x_ref, w_ref, out_ref):
    out_ref[...] = jnp.dot(
        x_ref[...].astype(jnp.bfloat16), w_ref[...],
        preferred_element_type=jnp.float32).astype(jnp.bfloat16)


def _inproj(x_flat, w2):
    M, D = x_flat.shape
    N = w2.shape[1]
    BM = 1024 if M % 1024 == 0 else M
    return pl.pallas_call(
        _inproj_kernel,
        out_shape=jax.ShapeDtypeStruct((M, N), jnp.bfloat16),
        grid=(M // BM,),
        in_specs=[
            pl.BlockSpec((BM, D), lambda i: (i, 0)),
            pl.BlockSpec((D, N), lambda i: (0, 0)),
        ],
        out_specs=pl.BlockSpec((BM, N), lambda i: (i, 0)),
        compiler_params=pltpu.CompilerParams(
            dimension_semantics=("arbitrary",),
        ),
        name="lstm_inproj",
    )(x_flat, w2)


def _lstm_kernel(xproj, len_ref, x_ref, wih_ref, whh_ref, b_ref, out_ref,
                 h_ref, c_ref):
    d = pl.program_id(0)
    t = pl.program_id(1)
    T = pl.num_programs(1)

    @pl.when(t == 0)
    def _():
        h_ref[...] = jnp.zeros_like(h_ref)
        c_ref[...] = jnp.zeros_like(c_ref)

    t_actual = jnp.where(d == 0, t, T - 1 - t)
    hd = h_ref.shape[1]
    if xproj:
        gx = jnp.dot(x_ref[0, 0], wih_ref[0],
                     preferred_element_type=jnp.float32)
    else:
        gx = x_ref[0, 0].astype(jnp.float32)
    g = (gx
         + jnp.dot(h_ref[...], whh_ref[0], preferred_element_type=jnp.float32)
         + b_ref[0])
    i = jax.nn.sigmoid(g[:, 0 * hd:1 * hd])
    f = jax.nn.sigmoid(g[:, 1 * hd:2 * hd])
    gg = jnp.tanh(g[:, 2 * hd:3 * hd])
    o = jax.nn.sigmoid(g[:, 3 * hd:4 * hd])
    c_new = f * c_ref[...] + i * gg
    h_new = o * jnp.tanh(c_new)
    m = len_ref[...] > t_actual
    h_ref[...] = jnp.where(m, h_new, h_ref[...])
    c_ref[...] = jnp.where(m, c_new, c_ref[...])
    out_ref[0] = jnp.where(m, h_new, 0.0)


def _lstm2_kernel(xproj, len_ref, x_ref, wih_ref, whh_ref, b_ref, out_ref,
                  h_ref, c_ref):
    d = pl.program_id(0)
    s = pl.program_id(1)
    S = pl.num_programs(1)
    T = 2 * S

    @pl.when(s == 0)
    def _():
        h_ref[...] = jnp.zeros_like(h_ref)
        c_ref[...] = jnp.zeros_like(c_ref)

    hd = h_ref.shape[1]
    fwd = d == 0
    t_first = jnp.where(fwd, 2 * s, T - 1 - 2 * s)
    t_second = jnp.where(fwd, 2 * s + 1, T - 2 - 2 * s)
    xb = x_ref[0]
    x_first = jnp.where(fwd, xb[0], xb[1])
    x_second = jnp.where(fwd, xb[1], xb[0])
    lenv = len_ref[...]

    def cell(xrow, t_actual):
        if xproj:
            gx = jnp.dot(xrow, wih_ref[0], preferred_element_type=jnp.float32)
        else:
            gx = xrow.astype(jnp.float32)
        g = (gx + jnp.dot(h_ref[...], whh_ref[0],
                          preferred_element_type=jnp.float32) + b_ref[0])
        i = jax.nn.sigmoid(g[:, 0 * hd:1 * hd])
        f = jax.nn.sigmoid(g[:, 1 * hd:2 * hd])
        gg = jnp.tanh(g[:, 2 * hd:3 * hd])
        o = jax.nn.sigmoid(g[:, 3 * hd:4 * hd])
        c_new = f * c_ref[...] + i * gg
        h_new = o * jnp.tanh(c_new)
        m = lenv > t_actual
        h_ref[...] = jnp.where(m, h_new, h_ref[...])
        c_ref[...] = jnp.where(m, c_new, c_ref[...])
        return jnp.where(m, h_new, 0.0)

    o_first = cell(x_first, t_first)
    o_second = cell(x_second, t_second)
    o_early = jnp.where(fwd, o_first, o_second)
    o_late = jnp.where(fwd, o_second, o_first)
    out_ref[...] = jnp.stack([o_early, o_late])


def _bilstm_layer(x_tm, lengths, wih_f, whh_f, b_f, wih_b, whh_b, b_b,
                  hoist_inproj):
    T, B, D = x_tm.shape
    HD4 = wih_f.shape[0]
    HD = HD4 // 4

    if hoist_inproj:
        wih2 = jnp.concatenate([wih_f.T, wih_b.T], axis=1).astype(jnp.bfloat16)
        gflat = _inproj(x_tm.reshape(T * B, D), wih2)
        xin = gflat.reshape(T, B, 2, HD4).transpose(2, 0, 1, 3)
        wih = jnp.zeros((2, 1, HD4), jnp.float32)
        DI = HD4
    else:
        xin = jnp.broadcast_to(x_tm[None], (2, T, B, D))
        wih = jnp.stack([wih_f.T, wih_b.T])
        DI = D

    whh = jnp.stack([whh_f.T, whh_b.T])
    b = jnp.stack([b_f[None, :], b_b[None, :]])
    len2 = lengths.astype(jnp.int32)[:, None]

    S = T // 2

    def p_idx(d, s):
        return jnp.where(d == 0, s, S - 1 - s)

    kern = functools.partial(_lstm2_kernel, not hoist_inproj)
    return pl.pallas_call(
        kern,
        out_shape=jax.ShapeDtypeStruct((T, B, 2 * HD), jnp.float32),
        grid=(2, S),
        in_specs=[
            pl.BlockSpec((B, 1), lambda d, s: (0, 0)),
            pl.BlockSpec((1, 2, B, DI), lambda d, s: (d, p_idx(d, s), 0, 0)),
            pl.BlockSpec((1, wih.shape[1], HD4), lambda d, s: (d, 0, 0)),
            pl.BlockSpec((1, HD, HD4), lambda d, s: (d, 0, 0)),
            pl.BlockSpec((1, 1, HD4), lambda d, s: (d, 0, 0)),
        ],
        out_specs=pl.BlockSpec((2, B, HD), lambda d, s: (p_idx(d, s), 0, d)),
        scratch_shapes=[
            pltpu.VMEM((B, HD), jnp.float32),
            pltpu.VMEM((B, HD), jnp.float32),
        ],
        compiler_params=pltpu.CompilerParams(
            dimension_semantics=("arbitrary", "arbitrary"),
        ),
        name="bilstm_layer",
    )(len2, xin, wih, whh, b)


def _decoder_kernel(K, A, n_chunk,
                    len_ref, dec_ref, mem_ref, mW_ref, wih_ref, whh_ref, b_ref,
                    qW_ref, M2_ref, v_ref, phW_ref, phb_ref, psW_ref, psb_ref,
                    hid_out, logit_out, align_out,
                    h_ref, c_ref, ctx_ref, awp_ref, awcp_ref, U_ref, pm_ref):
    t = pl.program_id(1)
    B, Tm, H = mem_ref.shape
    HD = h_ref.shape[1]
    pad = (K - 1) // 2

    @pl.when(t == 0)
    def _():
        h_ref[...] = jnp.zeros_like(h_ref)
        c_ref[...] = jnp.zeros_like(c_ref)
        ctx_ref[...] = jnp.zeros_like(ctx_ref)
        awp_ref[...] = jnp.zeros_like(awp_ref)
        awcp_ref[...] = jnp.zeros_like(awcp_ref)
        U_ref[...] = jnp.zeros_like(U_ref)
        for cj in range(n_chunk):
            sl0 = slice(cj * (B // n_chunk), (cj + 1) * (B // n_chunk))
            pm_ref[:, sl0, :] = jax.lax.dot_general(
                mW_ref[...].astype(jnp.bfloat16), mem_ref[sl0, :, :],
                (((1,), (2,)), ((), ())),
                preferred_element_type=jnp.float32).astype(jnp.bfloat16)

    cell_in = jnp.concatenate([dec_ref[0], ctx_ref[...]],
                              axis=1).astype(jnp.bfloat16)
    g = (jnp.dot(cell_in, wih_ref[...], preferred_element_type=jnp.float32)
         + jnp.dot(h_ref[...].astype(jnp.bfloat16), whh_ref[...],
                   preferred_element_type=jnp.float32)
         + b_ref[...])
    ig = jax.nn.sigmoid(g[:, 0 * HD:1 * HD])
    fg = jax.nn.sigmoid(g[:, 1 * HD:2 * HD])
    gg = jnp.tanh(g[:, 2 * HD:3 * HD])
    og = jax.nn.sigmoid(g[:, 3 * HD:4 * HD])
    c_new = fg * c_ref[...] + ig * gg
    h_new = og * jnp.tanh(c_new)
    h_ref[...] = h_new
    c_ref[...] = c_new

    for k in range(K):
        U_ref[k] = awp_ref[:, k:k + Tm].astype(jnp.bfloat16)
        U_ref[K + 1 + k] = awcp_ref[:, k:k + Tm].astype(jnp.bfloat16)

    q = jnp.dot(h_new, qW_ref[...], preferred_element_type=jnp.float32)
    q_t = q.T[:, :, None]
    v_col = v_ref[...].reshape(A)[:, None, None]

    e_parts = []
    cs = B // n_chunk
    for ci in range(n_chunk):
        sl = slice(ci * cs, (ci + 1) * cs)
        u_c = U_ref[:, sl, :]
        loc_c = jax.lax.dot_general(
            M2_ref[...], u_c, (((1,), (0,)), ((), ())),
            preferred_element_type=jnp.float32)
        x_c = loc_c + pm_ref[:, sl, :].astype(jnp.float32) + q_t[:, sl]
        e_parts.append(jnp.sum(v_col * jnp.tanh(x_c), axis=0))
    e = jnp.concatenate(e_parts, axis=0)

    tpos = jax.lax.broadcasted_iota(jnp.int32, (1, Tm), 1)
    e = jnp.where(tpos >= len_ref[...], -1e9, e)
    e = e - jnp.max(e, axis=1, keepdims=True)
    ex = jnp.exp(e)
    aw_new = ex / jnp.sum(ex, axis=1, keepdims=True)

    ctx_new = jnp.sum(aw_new[:, :, None] * mem_ref[...].astype(jnp.float32),
                      axis=1)
    ctx_ref[...] = ctx_new
    awp_ref[:, pad:pad + Tm] = aw_new
    awcp_ref[:, pad:pad + Tm] = awcp_ref[:, pad:pad + Tm] + aw_new

    hid = jnp.maximum(
        jnp.dot(jnp.concatenate([h_new, ctx_new], axis=1).astype(jnp.bfloat16),
                phW_ref[...],
                preferred_element_type=jnp.float32) + phb_ref[...], 0.0)
    logit = jnp.dot(hid, psW_ref[...], preferred_element_type=jnp.float32) + psb_ref[...]

    hid_out[0] = hid
    logit_out[0] = logit
    align_out[0] = aw_new


def _decoder(dec_seq, mem_len, memory, dec_Wih, dec_Whh, dec_b,
             q_W, m_W, loc_conv_W, loc_W, v_W, ph_W, ph_b, ps_W, ps_b):
    TD, B, EMB = dec_seq.shape
    Tm = memory.shape[1]
    H = memory.shape[2]
    HD = dec_Whh.shape[1]
    A = q_W.shape[0]
    NF, _, K = loc_conv_W.shape
    NSYM1 = ps_W.shape[0]
    pad = (K - 1) // 2
    TP = Tm + 2 * pad + (-(Tm + 2 * pad)) % 16

    M = jnp.einsum('af,fck->ack', loc_W, loc_conv_W)
    M2 = jnp.zeros((A, 2 * K + 2), jnp.float32)
    M2 = M2.at[:, 0:K].set(M[:, 0]).at[:, K + 1:2 * K + 1].set(M[:, 1])

    mem_bf16 = memory.astype(jnp.bfloat16)
    B2 = B
    NP = 1
    n_chunk = min(8, B2)

    kern = functools.partial(_decoder_kernel, K, A, n_chunk)
    hid, logit, align = pl.pallas_call(
        kern,
        out_shape=(
            jax.ShapeDtypeStruct((TD, B, H), jnp.float32),
            jax.ShapeDtypeStruct((TD, B, NSYM1), jnp.float32),
            jax.ShapeDtypeStruct((TD, B, Tm), jnp.float32),
        ),
        grid=(NP, TD),
        in_specs=[
            pl.BlockSpec((B2, 1), lambda p, t: (p, 0)),
            pl.BlockSpec((1, B2, EMB), lambda p, t: (t, p, 0)),
            pl.BlockSpec((B2, Tm, H), lambda p, t: (p, 0, 0)),
            pl.BlockSpec((A, H), lambda p, t: (0, 0)),
            pl.BlockSpec((EMB + H, 4 * HD), lambda p, t: (0, 0)),
            pl.BlockSpec((HD, 4 * HD), lambda p, t: (0, 0)),
            pl.BlockSpec((1, 4 * HD), lambda p, t: (0, 0)),
            pl.BlockSpec((HD, A), lambda p, t: (0, 0)),
            pl.BlockSpec((A, 2 * K + 2), lambda p, t: (0, 0)),
            pl.BlockSpec((1, A), lambda p, t: (0, 0)),
            pl.BlockSpec((H + H, H), lambda p, t: (0, 0)),
            pl.BlockSpec((1, H), lambda p, t: (0, 0)),
            pl.BlockSpec((H, NSYM1), lambda p, t: (0, 0)),
            pl.BlockSpec((1, NSYM1), lambda p, t: (0, 0)),
        ],
        out_specs=(
            pl.BlockSpec((1, B2, H), lambda p, t: (t, p, 0)),
            pl.BlockSpec((1, B2, NSYM1), lambda p, t: (t, p, 0)),
            pl.BlockSpec((1, B2, Tm), lambda p, t: (t, p, 0)),
        ),
        scratch_shapes=[
            pltpu.VMEM((B2, HD), jnp.float32),
            pltpu.VMEM((B2, HD), jnp.float32),
            pltpu.VMEM((B2, H), jnp.float32),
            pltpu.VMEM((B2, TP), jnp.float32),
            pltpu.VMEM((B2, TP), jnp.float32),
            pltpu.VMEM((2 * K + 2, B2, Tm), jnp.bfloat16),
            pltpu.VMEM((A, B2, Tm), jnp.bfloat16),
        ],
        compiler_params=pltpu.CompilerParams(
            dimension_semantics=("arbitrary", "arbitrary"),
            vmem_limit_bytes=61440000,
        ),
        name="decoder",
    )(mem_len.astype(jnp.int32)[:, None], dec_seq, mem_bf16, m_W,
      dec_Wih.T.astype(jnp.bfloat16), dec_Whh.T.astype(jnp.bfloat16),
      dec_b[None, :], q_W.T, M2.astype(jnp.bfloat16), v_W,
      ph_W.T.astype(jnp.bfloat16), ph_b[None, :], ps_W.T, ps_b[None, :])
    return hid, logit, align


def kernel(mel, mel_lengths, decoder_inputs, start_embedding, l1f_Wih, l1f_Whh,
           l1f_b, l1b_Wih, l1b_Whh, l1b_b, l2f_Wih, l2f_Whh, l2f_b, l2b_Wih,
           l2b_Whh, l2b_b, dec_Wih, dec_Whh, dec_b, q_W, m_W, loc_conv_W,
           loc_W, v_W, ph_W, ph_b, ps_W, ps_b):
    B, NMEL, T = mel.shape
    lengths = mel_lengths.astype(jnp.int32)

    x_tm = mel.transpose(2, 0, 1)
    cat1 = _bilstm_layer(x_tm, lengths, l1f_Wih, l1f_Whh, l1f_b,
                         l1b_Wih, l1b_Whh, l1b_b, False)
    x2 = cat1.transpose(1, 0, 2).reshape(B, T // 2, 2 * cat1.shape[2]) \
             .transpose(1, 0, 2)

    mem_len = jnp.ceil(lengths.astype(jnp.float32) / 2).astype(jnp.int32)
    memory = _bilstm_layer(x2, mem_len, l2f_Wih, l2f_Whh, l2f_b,
                           l2b_Wih, l2b_Whh, l2b_b, True)
    memory = memory.transpose(1, 0, 2)

    dec_seq = jnp.concatenate(
        [start_embedding[None], decoder_inputs.transpose(2, 0, 1)], axis=0)

    hid, logit, align = _decoder(dec_seq, mem_len, memory, dec_Wih, dec_Whh,
                                 dec_b, q_W, m_W, loc_conv_W, loc_W, v_W,
                                 ph_W, ph_b, ps_W, ps_b)
    return (hid.transpose(1, 0, 2), logit.transpose(1, 0, 2),
            align.transpose(1, 0, 2))
